# baseline (speedup 1.0000x reference)
import functools

import jax
import jax.numpy as jnp
from jax import lax
from jax.experimental import pallas as pl
from jax.experimental.pallas import tpu as pltpu
from jax.experimental.pallas import tpu_sc as plsc

N_NODES = 10000
N_EDGES = 160000
DIM = 256
HALF = DIM // 2
NUM_HOPS = 3
MIN_NORM = 1e-15
EPS = 1e-7

NC = 2
NS = 16
LANES = 16

CHUNK = 80
NB = 4
ND = 8
NCH = 128
EPT = NCH * CHUNK
E_PAD = EPT * NS
N_PAD = 10240
RPT = N_PAD // NS
NZ = RPT // CHUNK


def _pre_body(w_ref, o_ref):
    w = w_ref[...]
    d = w[:, 1:]
    y2 = jnp.sum(d * d, axis=1, keepdims=True)
    x0 = jnp.sqrt(jnp.clip(1.0 + y2, EPS, None))
    y_norm = jnp.clip(jnp.sqrt(y2), MIN_NORM, None)
    theta = jnp.clip(x0, 1.0 + EPS, None)
    r = jnp.log(theta + jnp.sqrt(theta * theta - 1.0))
    res = (r / y_norm) * d
    xt = jnp.concatenate([jnp.zeros_like(w[:, :1]), res], axis=1)
    o_ref[0] = xt[:, :HALF]
    o_ref[1] = xt[:, HALF:]


def _pre(weight):
    rows = 1000
    return pl.pallas_call(
        _pre_body,
        grid=(N_NODES // rows,),
        in_specs=[pl.BlockSpec((rows, DIM), lambda i: (i, 0))],
        out_specs=pl.BlockSpec((2, rows, HALF), lambda i: (0, i, 0)),
        out_shape=jax.ShapeDtypeStruct((2, N_NODES, HALF), jnp.float32),
    )(weight)


def _post_body(h_ref, o_ref):
    h = h_ref[...]
    acc = h[1] + h[2] + h[3]
    u = jnp.concatenate([acc[0], acc[1]], axis=1)
    d = u[:, 1:]
    x_norm = jnp.clip(jnp.sqrt(jnp.sum(d * d, axis=1, keepdims=True)),
                      MIN_NORM, None)
    sinh = 0.5 * (jnp.exp(x_norm) - jnp.exp(-x_norm))
    rest = sinh * d / x_norm
    y2 = jnp.sum(rest * rest, axis=1, keepdims=True)
    x0 = jnp.sqrt(jnp.clip(1.0 + y2, EPS, None))
    o_ref[...] = jnp.concatenate([x0, rest], axis=1)


def _post(hs):
    rows = 1000
    return pl.pallas_call(
        _post_body,
        grid=(N_NODES // rows,),
        in_specs=[pl.BlockSpec((NUM_HOPS + 1, 2, rows, HALF),
                               lambda i: (0, 0, i, 0))],
        out_specs=pl.BlockSpec((rows, DIM), lambda i: (i, 0)),
        out_shape=jax.ShapeDtypeStruct((N_NODES, DIM), jnp.float32),
    )(hs)


def _sc_body(h0, srcs, dsts, ws, out,
             src_sl, dst_sl, w_sl, bufs, isems, dsems, gsems, ssems, zsem,
             acc_sh):
    c = lax.axis_index("c")
    s = lax.axis_index("s")
    zeros16 = jnp.zeros((LANES,), jnp.float32)
    ebase = s * EPT

    def istart_sw(sl, ch):
        off = ebase + ch * CHUNK
        pltpu.async_copy(srcs.at[pl.ds(off, CHUNK)], src_sl.at[sl],
                         isems.at[sl])
        pltpu.async_copy(ws.at[pl.ds(off, CHUNK)], w_sl.at[sl],
                         isems.at[sl])

    def istart_d(sl, ch):
        off = ebase + ch * CHUNK
        pltpu.async_copy(dsts.at[pl.ds(off, CHUNK)], dst_sl.at[sl],
                         dsems.at[sl])

    def iwait(b4, b8, ch):
        off = ebase + ch * CHUNK
        pltpu.make_async_copy(srcs.at[pl.ds(off, CHUNK)], src_sl.at[b4],
                              isems.at[b4]).wait()
        pltpu.make_async_copy(ws.at[pl.ds(off, CHUNK)], w_sl.at[b4],
                              isems.at[b4]).wait()
        pltpu.make_async_copy(dsts.at[pl.ds(off, CHUNK)], dst_sl.at[b8],
                              dsems.at[b8]).wait()

    def scale_chunk(db, wsl):
        buf = bufs.at[db]
        wrow = w_sl.at[wsl]

        def body16(e16, _):
            wv = wrow[pl.ds(e16 * LANES, LANES)]
            for k in range(LANES):
                w = wv[k]
                e = e16 * LANES + k
                for j in range(HALF // LANES):
                    sl = buf[e, pl.ds(j * LANES, LANES)]
                    buf[e, pl.ds(j * LANES, LANES)] = sl * w
            return 0
        lax.fori_loop(0, CHUNK // LANES, body16, 0)

    @pl.when(s < NS - 1)
    def _():
        pltpu.sync_copy(h0.at[c, pl.ds(s * RPT, RPT)],
                        out.at[0, c, pl.ds(s * RPT, RPT)])

    @pl.when(s == NS - 1)
    def _():
        last = N_NODES - (NS - 1) * RPT
        pltpu.sync_copy(h0.at[c, pl.ds((NS - 1) * RPT, last)],
                        out.at[0, c, pl.ds((NS - 1) * RPT, last)])

    def hop_body(hop, _):
        hsrc = out.at[hop, c]

        def gstart(db, i):
            pltpu.async_copy(hsrc.at[src_sl.at[db]], bufs.at[db],
                             gsems.at[db])

        def gwait(db):
            pltpu.make_async_copy(hsrc.at[src_sl.at[db]], bufs.at[db],
                                  gsems.at[db]).wait()

        def sstart(db, b8):
            pltpu.async_copy(bufs.at[db], acc_sh.at[dst_sl.at[b8]],
                             ssems.at[db], add=True)

        def swait(db, b8):
            pltpu.make_async_copy(bufs.at[db], acc_sh.at[dst_sl.at[b8]],
                                  ssems.at[db]).wait()

        def zbody(e, _):
            for j in range(HALF // LANES):
                bufs[0, e, pl.ds(j * LANES, LANES)] = zeros16
            return 0
        lax.fori_loop(0, CHUNK, zbody, 0)
        for z in range(NZ):
            pltpu.async_copy(bufs.at[0],
                             acc_sh.at[pl.ds(s * RPT + z * CHUNK, CHUNK)],
                             zsem)
        for b in range(NB):
            istart_sw(b, b)
        for b in range(ND):
            istart_d(b, b)
        for z in range(NZ):
            pltpu.make_async_copy(
                bufs.at[0], acc_sh.at[pl.ds(s * RPT + z * CHUNK, CHUNK)],
                zsem).wait()
        plsc.subcore_barrier()

        def step(i, _):
            b4 = lax.rem(i, NB)
            b8 = lax.rem(i, ND)
            pb8 = lax.rem(i + NB, ND)

            @pl.when(i >= NB)
            def _():
                swait(b4, pb8)

                @pl.when(i + NB < NCH)
                def _():
                    istart_d(pb8, i + NB)

            @pl.when(i < NCH)
            def _():
                iwait(b4, b8, i)
                gstart(b4, i)

            j = i - 2
            jb4 = lax.rem(j + NB, NB)
            jb8 = lax.rem(j + ND, ND)

            @pl.when(jnp.logical_and(j >= 0, j < NCH))
            def _():
                gwait(jb4)
                scale_chunk(jb4, jb4)
                sstart(jb4, jb8)

                @pl.when(j + NB < NCH)
                def _():
                    istart_sw(jb4, j + NB)
            return 0

        lax.fori_loop(0, NCH + NB, step, 0)
        plsc.subcore_barrier()

        pltpu.sync_copy(acc_sh.at[pl.ds(s * RPT, RPT)],
                        out.at[hop + 1, c, pl.ds(s * RPT, RPT)])
        plsc.subcore_barrier()
        return 0

    lax.fori_loop(0, NUM_HOPS, hop_body, 0)


def _spmm(xt2, srcs, dsts, ws):
    mesh = plsc.VectorSubcoreMesh(core_axis_name="c", subcore_axis_name="s")
    f = functools.partial(
        pl.kernel,
        mesh=mesh,
        out_type=jax.ShapeDtypeStruct((NUM_HOPS + 1, 2, N_PAD, HALF),
                                      jnp.float32),
        scratch_types=[
            pltpu.VMEM((NB, CHUNK), jnp.int32),
            pltpu.VMEM((ND, CHUNK), jnp.int32),
            pltpu.VMEM((NB, CHUNK), jnp.float32),
            pltpu.VMEM((NB, CHUNK, HALF), jnp.float32),
            pltpu.SemaphoreType.DMA((NB,)),
            pltpu.SemaphoreType.DMA((ND,)),
            pltpu.SemaphoreType.DMA((NB,)),
            pltpu.SemaphoreType.DMA((NB,)),
            pltpu.SemaphoreType.DMA,
            pltpu.VMEM_SHARED((N_PAD, HALF), jnp.float32),
        ],
    )(_sc_body)
    return f(xt2, srcs, dsts, ws)


def kernel(weight, edge_index, edge_weight):
    xt2 = _pre(weight)
    pad = E_PAD - N_EDGES
    srcs = jnp.concatenate([edge_index[0], jnp.zeros((pad,), jnp.int32)])
    dsts = jnp.concatenate([edge_index[1], jnp.zeros((pad,), jnp.int32)])
    ws = jnp.concatenate([edge_weight, jnp.zeros((pad,), jnp.float32)])
    hs = _spmm(xt2, srcs, dsts, ws)
    return _post(hs)
